# baseline (speedup 1.0000x reference)
import functools

import jax
import jax.numpy as jnp
from jax import lax
from jax.experimental import pallas as pl
from jax.experimental.pallas import tpu as pltpu
from jax.experimental.pallas import tpu_sc as plsc

H = 16
KM = 4
DIM = 2
IDX_CHUNK = 128
PW = 80
OW = 128


def _sc_gather(tbl, r1g, r2g):
    n = r1g.shape[0] * IDX_CHUNK
    info = plsc.get_sparse_core_info()
    nc, ns = info.num_cores, info.num_subcores
    nw = nc * ns
    bpw = n // nw
    nch = bpw // IDX_CHUNK

    mesh = plsc.VectorSubcoreMesh(core_axis_name="c", subcore_axis_name="s")
    f32 = jnp.float32
    out_type = [
        jax.ShapeDtypeStruct((n, OW), f32),
        jax.ShapeDtypeStruct((n, OW), f32),
    ]
    scratch_types = [
        pltpu.VMEM((nch, IDX_CHUNK), jnp.int32),
        pltpu.VMEM((nch, IDX_CHUNK), jnp.int32),
        pltpu.VMEM((bpw, PW), f32),
        pltpu.VMEM((bpw, PW), f32),
        pltpu.SemaphoreType.DMA,
    ]

    @functools.partial(pl.kernel, mesh=mesh, out_type=out_type,
                       scratch_types=scratch_types,
                       compiler_params=pltpu.CompilerParams(
                           use_tc_tiling_on_sc=False))
    def k(tref, r1h, r2h, oa, ob, i1, i2, bufa, bufb, sem):
        wid = lax.axis_index("s") * nc + lax.axis_index("c")
        base = wid * bpw
        pltpu.sync_copy(r1h.at[pl.ds(wid * nch, nch), :], i1)
        pltpu.sync_copy(r2h.at[pl.ds(wid * nch, nch), :], i2)
        handles = []
        for idxv, buf in ((i1, bufa), (i2, bufb)):
            for c in range(nch):
                handles.append(pltpu.async_copy(
                    tref.at[idxv.at[c]],
                    buf.at[pl.ds(c * IDX_CHUNK, IDX_CHUNK), :],
                    sem))
        for hdl in handles:
            hdl.wait()
        pltpu.sync_copy(bufa, oa.at[pl.ds(base, bpw), pl.ds(0, PW)])
        pltpu.sync_copy(bufb, ob.at[pl.ds(base, bpw), pl.ds(0, PW)])

    return k(tbl, r1g, r2g)


def _tc_body(ga_r, gb_r, out_r):
    def unpack(g):
        return (g[:, :H], g[:, H:2 * H],
                g[:, 2 * H:3 * H], g[:, 3 * H:4 * H],
                g[:, 64:64 + 2 * KM],
                g[:, 72:72 + KM],
                g[:, 76:77])

    def side(va0, va1, b1f, w2f, b2f, mxm, mmm):
        u = jnp.zeros_like(b1f)
        for k in range(KM):
            x0 = mxm[:, 2 * k:2 * k + 1]
            x1 = mxm[:, 2 * k + 1:2 * k + 2]
            z = va0 * x0 + va1 * x1 + b1f
            u = u + mmm[:, k:k + 1] * jnp.tanh(z)
        s = jnp.sum(u * w2f, axis=1, keepdims=True)
        return s + b2f * jnp.sum(mmm, axis=1, keepdims=True)

    a0, a1, b1a, w2a, mxa, mma, b2a = unpack(ga_r[...])
    b0, b1v, b1b, w2b, mxb, mmb, b2b = unpack(gb_r[...])
    s1 = side(a0, a1, b1a, w2a, b2a, mxb, mmb)
    s2 = side(b0, b1v, b1b, w2b, b2b, mxa, mma)
    out_r[...] = (s1 + s2)[:, 0]


def _tc_compute(ga, gb):
    n = ga.shape[0]
    bt = 4096
    return pl.pallas_call(
        _tc_body,
        grid=(n // bt,),
        in_specs=[pl.BlockSpec((bt, OW), lambda i: (i, 0)),
                  pl.BlockSpec((bt, OW), lambda i: (i, 0))],
        out_specs=pl.BlockSpec((bt,), lambda i: (i,)),
        out_shape=jax.ShapeDtypeStruct((n,), jnp.float32),
    )(ga, gb)


def kernel(pairs, W1, b1, w2, b2, mx, mm):
    v = W1.shape[0]
    r1 = pairs[:, 0].astype(jnp.int32).reshape(-1, IDX_CHUNK)
    r2 = pairs[:, 1].astype(jnp.int32).reshape(-1, IDX_CHUNK)
    tbl = jnp.concatenate(
        [jnp.swapaxes(W1, 1, 2).reshape(v, 2 * H), b1, w2,
         mx.reshape(v, KM * DIM), mm, b2[:, None],
         jnp.zeros((v, PW - 4 * H - KM * DIM - KM - 1), jnp.float32)],
        axis=1)
    ga, gb = _sc_gather(tbl, r1, r2)
    return _tc_compute(ga, gb)


# speedup vs baseline: 1.7149x; 1.7149x over previous
import functools

import jax
import jax.numpy as jnp
from jax import lax
from jax.experimental import pallas as pl
from jax.experimental.pallas import tpu as pltpu
from jax.experimental.pallas import tpu_sc as plsc

H = 16
KM = 4
DIM = 2
IDX_CHUNK = 128
OW = 128


def _sc_gather(w1f, b1, w2, me, r1g, r2g):
    n = r1g.shape[0] * IDX_CHUNK
    info = plsc.get_sparse_core_info()
    nc, ns = info.num_cores, info.num_subcores
    nw = nc * ns
    bpw = n // nw
    nch = bpw // IDX_CHUNK

    mesh = plsc.VectorSubcoreMesh(core_axis_name="c", subcore_axis_name="s")
    f32 = jnp.float32
    out_type = [
        jax.ShapeDtypeStruct((n, OW), f32),
        jax.ShapeDtypeStruct((n, OW), f32),
    ]
    scratch_types = [
        pltpu.VMEM((nch, IDX_CHUNK), jnp.int32),
        pltpu.VMEM((nch, IDX_CHUNK), jnp.int32),
        pltpu.VMEM((bpw, 2 * H), f32),
        pltpu.VMEM((bpw, H), f32),
        pltpu.VMEM((bpw, H), f32),
        pltpu.VMEM((bpw, H), f32),
        pltpu.VMEM((bpw, 2 * H), f32),
        pltpu.VMEM((bpw, H), f32),
        pltpu.VMEM((bpw, H), f32),
        pltpu.VMEM((bpw, H), f32),
        pltpu.SemaphoreType.DMA,
    ]

    @functools.partial(pl.kernel, mesh=mesh, out_type=out_type,
                       scratch_types=scratch_types,
                       compiler_params=pltpu.CompilerParams(
                           use_tc_tiling_on_sc=False))
    def k(tw1, tb1, tw2, tme, r1h, r2h, oa, ob,
          i1, i2, bw1a, bb1a, bw2a, bmea, bw1b, bb1b, bw2b, bmeb, sem):
        wid = lax.axis_index("s") * nc + lax.axis_index("c")
        base = wid * bpw
        pltpu.sync_copy(r1h.at[pl.ds(wid * nch, nch), :], i1)
        pltpu.sync_copy(r2h.at[pl.ds(wid * nch, nch), :], i2)
        handles = []
        for idxv, bufs in ((i1, (bw1a, bb1a, bw2a, bmea)),
                           (i2, (bw1b, bb1b, bw2b, bmeb))):
            for tbl, buf in zip((tw1, tb1, tw2, tme), bufs):
                for c in range(nch):
                    handles.append(pltpu.async_copy(
                        tbl.at[idxv.at[c]],
                        buf.at[pl.ds(c * IDX_CHUNK, IDX_CHUNK), :],
                        sem))
        for hdl in handles:
            hdl.wait()
        for out, bufs in ((oa, (bw1a, bb1a, bw2a, bmea)),
                          (ob, (bw1b, bb1b, bw2b, bmeb))):
            col = 0
            for buf in bufs:
                w = buf.shape[1]
                pltpu.sync_copy(buf, out.at[pl.ds(base, bpw), pl.ds(col, w)])
                col += w

    return k(w1f, b1, w2, me, r1g, r2g)


def _tc_body(ga_r, gb_r, out_r):
    def parts(g):
        return (g[:, :H], g[:, H:2 * H],
                g[:, 2 * H:3 * H], g[:, 3 * H:4 * H],
                g[:, 64:64 + 2 * KM],
                g[:, 72:72 + KM],
                g[:, 76:77])

    a0, a1, ab1, aw2, amx, amm, ab2 = parts(ga_r[...])
    b0, b1v, bb1, bw2, bmx, bmm, bb2 = parts(gb_r[...])
    bt = a0.shape[0]
    cat = lambda ps: jnp.concatenate(ps, axis=1)
    brd = lambda col: jnp.broadcast_to(col, (bt, H))
    w0 = cat([a0] * KM + [b0] * KM)
    w1 = cat([a1] * KM + [b1v] * KM)
    bc = cat([ab1] * KM + [bb1] * KM)
    ww = cat([aw2] * KM + [bw2] * KM)
    x0 = cat([brd(bmx[:, 2 * k:2 * k + 1]) for k in range(KM)]
             + [brd(amx[:, 2 * k:2 * k + 1]) for k in range(KM)])
    x1 = cat([brd(bmx[:, 2 * k + 1:2 * k + 2]) for k in range(KM)]
             + [brd(amx[:, 2 * k + 1:2 * k + 2]) for k in range(KM)])
    m = cat([brd(bmm[:, k:k + 1]) for k in range(KM)]
            + [brd(amm[:, k:k + 1]) for k in range(KM)])
    t = jnp.tanh(w0 * x0 + w1 * x1 + bc)
    s = jnp.sum(ww * m * t, axis=1)
    smma = jnp.sum(amm, axis=1)
    smmb = jnp.sum(bmm, axis=1)
    out_r[...] = s + ab2[:, 0] * smmb + bb2[:, 0] * smma


def _tc_compute(ga, gb):
    n = ga.shape[0]
    bt = 2048
    return pl.pallas_call(
        _tc_body,
        grid=(n // bt,),
        in_specs=[pl.BlockSpec((bt, OW), lambda i: (i, 0)),
                  pl.BlockSpec((bt, OW), lambda i: (i, 0))],
        out_specs=pl.BlockSpec((bt,), lambda i: (i,)),
        out_shape=jax.ShapeDtypeStruct((n,), jnp.float32),
    )(ga, gb)


def kernel(pairs, W1, b1, w2, b2, mx, mm):
    v = W1.shape[0]
    r1 = pairs[:, 0].astype(jnp.int32).reshape(-1, IDX_CHUNK)
    r2 = pairs[:, 1].astype(jnp.int32).reshape(-1, IDX_CHUNK)
    w1f = jnp.swapaxes(W1, 1, 2).reshape(v, 2 * H)
    me = jnp.concatenate(
        [mx.reshape(v, KM * DIM), mm, b2[:, None],
         jnp.zeros((v, H - KM * DIM - KM - 1), jnp.float32)], axis=1)
    ga, gb = _sc_gather(w1f, b1, w2, me, r1, r2)
    return _tc_compute(ga, gb)
